# baseline (speedup 1.0000x reference)
import jax
import jax.numpy as jnp
from jax.experimental import pallas as pl

_VOCAB = 1000
_ROWS_PER_BLK = 512


def _onehot_block(x_ref, o_ref):
    idx = x_ref[...]
    iota = jax.lax.broadcasted_iota(jnp.int32, (_ROWS_PER_BLK, _VOCAB), 1)
    o_ref[...] = (idx == iota).astype(jnp.float32)


def kernel(x, W):
    B, H = x.shape
    N = B * H
    x2 = x.reshape(N, 1).astype(jnp.int32)
    G = N // _ROWS_PER_BLK
    out = pl.pallas_call(
        _onehot_block,
        grid=(G,),
        in_specs=[pl.BlockSpec((_ROWS_PER_BLK, 1), lambda i: (i, 0))],
        out_specs=pl.BlockSpec((_ROWS_PER_BLK, _VOCAB), lambda i: (i, 0)),
        out_shape=jax.ShapeDtypeStruct((N, _VOCAB), jnp.float32),
    )(x2)
    return out.reshape(B, H, _VOCAB)


# speedup vs baseline: 1.6779x; 1.6779x over previous
import jax
import jax.numpy as jnp
from jax.experimental import pallas as pl

_VOCAB = 1000
_ROWS_PER_BLK = 64


def _onehot_block(x_ref, o_ref):
    idx = x_ref[...]
    iota = jax.lax.broadcasted_iota(jnp.int32, o_ref.shape, 2)
    o_ref[...] = (idx == iota).astype(jnp.float32)


def kernel(x, W):
    B, H = x.shape
    x3 = x.reshape(B, H, 1).astype(jnp.int32)
    G = B // _ROWS_PER_BLK
    out = pl.pallas_call(
        _onehot_block,
        grid=(G,),
        in_specs=[pl.BlockSpec((_ROWS_PER_BLK, H, 1), lambda i: (i, 0, 0))],
        out_specs=pl.BlockSpec((_ROWS_PER_BLK, H, _VOCAB), lambda i: (i, 0, 0)),
        out_shape=jax.ShapeDtypeStruct((B, H, _VOCAB), jnp.float32),
    )(x3)
    return out
